# Initial kernel scaffold; baseline (speedup 1.0000x reference)
import functools

import jax
import jax.numpy as jnp
from jax import lax
from jax.experimental import pallas as pl
from jax.experimental.pallas import tpu as pltpu
from jax.experimental.pallas import tpu_sc as plsc

NC = 2
NS = 16
NW = NC * NS
CHUNK = 128


def _sc_mesh():
    return plsc.VectorSubcoreMesh(
        core_axis_name="c", subcore_axis_name="s", num_cores=NC, num_subcores=NS
    )


@functools.lru_cache(maxsize=None)
def _make_deg_kernel(ch, acc_rows, rps):
    @functools.partial(
        pl.kernel,
        out_type=jax.ShapeDtypeStruct((NC, acc_rows, 16), jnp.float32),
        mesh=_sc_mesh(),
        scratch_types=[
            pltpu.VMEM((ch, CHUNK), jnp.int32),
            pltpu.VMEM((CHUNK, 16), jnp.float32),
            pltpu.VMEM_SHARED((acc_rows, 16), jnp.float32),
            pltpu.SemaphoreType.DMA,
        ],
    )
    def deg_kernel(zeros_hbm, dst_hbm, out_hbm, dst_v, ones_v, acc_sh, sem):
        c = lax.axis_index("c")
        s = lax.axis_index("s")
        w = c * NS + s
        rbase = s * rps
        pltpu.async_copy(zeros_hbm, acc_sh.at[pl.ds(rbase, rps)], sem)
        @pl.loop(0, CHUNK)
        def _(i):
            ones_v[i, :] = jnp.full((16,), 1.0, jnp.float32)

        pltpu.sync_copy(dst_hbm.at[w], dst_v)
        sem.wait()
        plsc.subcore_barrier()

        @pl.loop(0, ch)
        def _(j):
            pltpu.sync_copy(ones_v, acc_sh.at[dst_v.at[j]], add=True)

        plsc.subcore_barrier()
        pltpu.sync_copy(
            acc_sh.at[pl.ds(rbase, rps)], out_hbm.at[c, pl.ds(rbase, rps)]
        )

    return deg_kernel


@functools.lru_cache(maxsize=None)
def _make_segsum_kernel(d, ch, acc_rows, rps):
    @functools.partial(
        pl.kernel,
        out_type=jax.ShapeDtypeStruct((NC, acc_rows, d), jnp.float32),
        mesh=_sc_mesh(),
        scratch_types=[
            pltpu.VMEM((ch, CHUNK), jnp.int32),
            pltpu.VMEM((ch, CHUNK), jnp.int32),
            pltpu.VMEM((CHUNK, d), jnp.float32),
            pltpu.VMEM((CHUNK, d), jnp.float32),
            pltpu.VMEM_SHARED((acc_rows, d), jnp.float32),
            pltpu.SemaphoreType.DMA,
            pltpu.SemaphoreType.DMA,
        ],
    )
    def segsum_kernel(
        table_hbm, zeros_hbm, src_hbm, dst_hbm, out_hbm,
        src_v, dst_v, rows_a, rows_b, acc_sh, sem_a, sem_b,
    ):
        c = lax.axis_index("c")
        s = lax.axis_index("s")
        w = c * NS + s
        rbase = s * rps
        pltpu.async_copy(zeros_hbm, acc_sh.at[pl.ds(rbase, rps)], sem_a)
        pltpu.sync_copy(src_hbm.at[w], src_v)
        pltpu.sync_copy(dst_hbm.at[w], dst_v)
        sem_a.wait()
        plsc.subcore_barrier()

        pltpu.async_copy(table_hbm.at[src_v.at[0]], rows_a, sem_a)
        pltpu.async_copy(table_hbm.at[src_v.at[1]], rows_b, sem_b)

        @pl.loop(0, ch, step=2)
        def _(j):
            sem_a.wait()
            pltpu.sync_copy(rows_a, acc_sh.at[dst_v.at[j]], add=True)

            @pl.when(j + 2 < ch)
            def _():
                pltpu.async_copy(table_hbm.at[src_v.at[j + 2]], rows_a, sem_a)

            sem_b.wait()
            pltpu.sync_copy(rows_b, acc_sh.at[dst_v.at[j + 1]], add=True)

            @pl.when(j + 3 < ch)
            def _():
                pltpu.async_copy(table_hbm.at[src_v.at[j + 3]], rows_b, sem_b)

        plsc.subcore_barrier()
        pltpu.sync_copy(
            acc_sh.at[pl.ds(rbase, rps)], out_hbm.at[c, pl.ds(rbase, rps)]
        )

    return segsum_kernel


def _matmul(x, w):
    m, k = x.shape
    k2, n = w.shape

    def body(x_ref, w_ref, o_ref):
        o_ref[...] = jnp.dot(
            x_ref[...], w_ref[...], preferred_element_type=jnp.float32
        )

    return pl.pallas_call(
        body, out_shape=jax.ShapeDtypeStruct((m, n), jnp.float32)
    )(x, w)


def _dinv_and_scale(degp, xw):
    n, hid = xw.shape

    def body(degp_ref, xw_ref, dinv_ref, xws_ref):
        deg = degp_ref[0, :n, 0:1] + degp_ref[1, :n, 0:1] + 1.0
        dinv = lax.rsqrt(deg)
        dinv_ref[...] = dinv
        xws_ref[...] = xw_ref[...] * dinv

    return pl.pallas_call(
        body,
        out_shape=(
            jax.ShapeDtypeStruct((n, 1), jnp.float32),
            jax.ShapeDtypeStruct((n, hid), jnp.float32),
        ),
    )(degp, xw)


def _combine_relu_mm(accp, xw, dinv, b, w2):
    n, hid = xw.shape
    cls = w2.shape[1]

    def body(accp_ref, xw_ref, dinv_ref, b_ref, w2_ref, h2_ref, xws2_ref):
        dv = dinv_ref[...]
        agg = accp_ref[0, :n, :] + accp_ref[1, :n, :]
        h = dv * agg + (dv * dv) * xw_ref[...] + b_ref[...]
        h = jnp.maximum(h, 0.0)
        h2 = jnp.dot(h, w2_ref[...], preferred_element_type=jnp.float32)
        h2_ref[...] = h2
        xws2_ref[...] = h2 * dv

    return pl.pallas_call(
        body,
        out_shape=(
            jax.ShapeDtypeStruct((n, cls), jnp.float32),
            jax.ShapeDtypeStruct((n, cls), jnp.float32),
        ),
    )(accp, xw, dinv, b, w2)


def _combine_logsoftmax(accp, h2, dinv, b):
    n, cls = h2.shape

    def body(accp_ref, h2_ref, dinv_ref, b_ref, o_ref):
        dv = dinv_ref[...]
        agg = accp_ref[0, :n, :] + accp_ref[1, :n, :]
        z = dv * agg + (dv * dv) * h2_ref[...] + b_ref[...]
        m = jnp.max(z, axis=1, keepdims=True)
        shifted = z - m
        lse = jnp.log(jnp.sum(jnp.exp(shifted), axis=1, keepdims=True))
        o_ref[...] = shifted - lse

    return pl.pallas_call(
        body, out_shape=jax.ShapeDtypeStruct((n, cls), jnp.float32)
    )(accp, h2, dinv, b)


def kernel(x, edge_index, W1, b1, W2, b2):
    n, f_in = x.shape
    e = edge_index.shape[1]
    hid = W1.shape[1]
    cls = W2.shape[1]

    per_block = NW * CHUNK
    ch = -(-e // per_block)
    ch += ch % 2
    epad = ch * per_block
    rps = -(-(n + 1) // (NS * 8)) * 8
    acc_rows = rps * NS

    src = edge_index[0]
    dst = edge_index[1]
    pad = epad - e
    srcp = jnp.concatenate([src, jnp.zeros((pad,), jnp.int32)]).reshape(
        NW, ch, CHUNK
    )
    dstp = jnp.concatenate(
        [dst, jnp.full((pad,), n, jnp.int32)]
    ).reshape(NW, ch, CHUNK)

    zeros16 = jnp.zeros((rps, 16), jnp.float32)
    zeros_cls = jnp.zeros((rps, cls), jnp.float32)

    degp = _make_deg_kernel(ch, acc_rows, rps)(zeros16, dstp)
    xw1 = _matmul(x, W1)
    dinv, xws1 = _dinv_and_scale(degp, xw1)
    acc1 = _make_segsum_kernel(hid, ch, acc_rows, rps)(
        xws1, zeros16, srcp, dstp
    )
    h2, xws2 = _combine_relu_mm(acc1, xw1, dinv, b1.reshape(1, hid), W2)
    acc2 = _make_segsum_kernel(cls, ch, acc_rows, rps)(
        xws2, zeros_cls, srcp, dstp
    )
    return _combine_logsoftmax(acc2, h2, dinv, b2.reshape(1, cls))


# speedup vs baseline: 22.1236x; 22.1236x over previous
import functools

import jax
import jax.numpy as jnp
from jax import lax
from jax.experimental import pallas as pl
from jax.experimental.pallas import tpu as pltpu
from jax.experimental.pallas import tpu_sc as plsc

NC = 2
NS = 16
NW = NC * NS
CHUNK = 128


def _sc_mesh():
    return plsc.VectorSubcoreMesh(
        core_axis_name="c", subcore_axis_name="s", num_cores=NC, num_subcores=NS
    )


@functools.lru_cache(maxsize=None)
def _make_deg_kernel(ch, acc_rows, rps):
    @functools.partial(
        pl.kernel,
        out_type=jax.ShapeDtypeStruct((NC, acc_rows, 16), jnp.float32),
        mesh=_sc_mesh(),
        compiler_params=pltpu.CompilerParams(use_tc_tiling_on_sc=False),
        scratch_types=[
            pltpu.VMEM((ch, CHUNK), jnp.int32),
            pltpu.VMEM((CHUNK, 16), jnp.float32),
            pltpu.VMEM_SHARED((acc_rows, 16), jnp.float32),
            pltpu.SemaphoreType.DMA,
        ],
    )
    def deg_kernel(zeros_hbm, dst_hbm, out_hbm, dst_v, ones_v, acc_sh, sem):
        c = lax.axis_index("c")
        s = lax.axis_index("s")
        w = c * NS + s
        rbase = s * rps
        zcp = pltpu.async_copy(zeros_hbm, acc_sh.at[pl.ds(rbase, rps)], sem)
        @pl.loop(0, CHUNK)
        def _(i):
            ones_v[i, :] = jnp.full((16,), 1.0, jnp.float32)

        pltpu.sync_copy(dst_hbm.at[w], dst_v)
        zcp.wait()
        plsc.subcore_barrier()

        @pl.loop(0, ch)
        def _(j):
            pltpu.sync_copy(ones_v, acc_sh.at[dst_v.at[j]], add=True)

        plsc.subcore_barrier()
        pltpu.sync_copy(
            acc_sh.at[pl.ds(rbase, rps)], out_hbm.at[c, pl.ds(rbase, rps)]
        )

    return deg_kernel


@functools.lru_cache(maxsize=None)
def _make_segsum_kernel(d, ch, acc_rows, rps):
    @functools.partial(
        pl.kernel,
        out_type=jax.ShapeDtypeStruct((NC, acc_rows, d), jnp.float32),
        mesh=_sc_mesh(),
        compiler_params=pltpu.CompilerParams(use_tc_tiling_on_sc=False),
        scratch_types=[
            pltpu.VMEM((ch, CHUNK), jnp.int32),
            pltpu.VMEM((ch, CHUNK), jnp.int32),
            pltpu.VMEM((CHUNK, d), jnp.float32),
            pltpu.VMEM((CHUNK, d), jnp.float32),
            pltpu.VMEM_SHARED((acc_rows, d), jnp.float32),
            pltpu.SemaphoreType.DMA,
            pltpu.SemaphoreType.DMA,
        ],
    )
    def segsum_kernel(
        table_hbm, zeros_hbm, src_hbm, dst_hbm, out_hbm,
        src_v, dst_v, rows_a, rows_b, acc_sh, sem_a, sem_b,
    ):
        c = lax.axis_index("c")
        s = lax.axis_index("s")
        w = c * NS + s
        rbase = s * rps
        zcp = pltpu.async_copy(zeros_hbm, acc_sh.at[pl.ds(rbase, rps)], sem_a)
        pltpu.sync_copy(src_hbm.at[w], src_v)
        pltpu.sync_copy(dst_hbm.at[w], dst_v)
        zcp.wait()
        plsc.subcore_barrier()

        pltpu.async_copy(table_hbm.at[src_v.at[0]], rows_a, sem_a)
        pltpu.async_copy(table_hbm.at[src_v.at[1]], rows_b, sem_b)

        @pl.loop(0, ch, step=2)
        def _(j):
            pltpu.make_async_copy(
                table_hbm.at[src_v.at[j]], rows_a, sem_a
            ).wait()
            pltpu.sync_copy(rows_a, acc_sh.at[dst_v.at[j]], add=True)

            @pl.when(j + 2 < ch)
            def _():
                pltpu.async_copy(table_hbm.at[src_v.at[j + 2]], rows_a, sem_a)

            pltpu.make_async_copy(
                table_hbm.at[src_v.at[j + 1]], rows_b, sem_b
            ).wait()
            pltpu.sync_copy(rows_b, acc_sh.at[dst_v.at[j + 1]], add=True)

            @pl.when(j + 3 < ch)
            def _():
                pltpu.async_copy(table_hbm.at[src_v.at[j + 3]], rows_b, sem_b)

        plsc.subcore_barrier()
        pltpu.sync_copy(
            acc_sh.at[pl.ds(rbase, rps)], out_hbm.at[c, pl.ds(rbase, rps)]
        )

    return segsum_kernel


def _matmul(x, w):
    m, k = x.shape
    k2, n = w.shape

    def body(x_ref, w_ref, o_ref):
        o_ref[...] = jnp.dot(
            x_ref[...], w_ref[...], preferred_element_type=jnp.float32
        )

    return pl.pallas_call(
        body, out_shape=jax.ShapeDtypeStruct((m, n), jnp.float32)
    )(x, w)


def _dinv_and_scale(degp, xw):
    n, hid = xw.shape

    def body(degp_ref, xw_ref, dinv_ref, xws_ref):
        deg = degp_ref[0, :n, 0:1] + degp_ref[1, :n, 0:1] + 1.0
        dinv = lax.rsqrt(deg)
        dinv_ref[...] = dinv
        xws_ref[...] = xw_ref[...] * dinv

    return pl.pallas_call(
        body,
        out_shape=(
            jax.ShapeDtypeStruct((n, 1), jnp.float32),
            jax.ShapeDtypeStruct((n, hid), jnp.float32),
        ),
    )(degp, xw)


def _combine_relu_mm(accp, xw, dinv, b, w2):
    n, hid = xw.shape
    cls = w2.shape[1]

    def body(accp_ref, xw_ref, dinv_ref, b_ref, w2_ref, h2_ref, xws2_ref):
        dv = dinv_ref[...]
        agg = accp_ref[0, :n, :] + accp_ref[1, :n, :]
        h = dv * agg + (dv * dv) * xw_ref[...] + b_ref[...]
        h = jnp.maximum(h, 0.0)
        h2 = jnp.dot(h, w2_ref[...], preferred_element_type=jnp.float32)
        h2_ref[...] = h2
        xws2_ref[...] = h2 * dv

    return pl.pallas_call(
        body,
        out_shape=(
            jax.ShapeDtypeStruct((n, cls), jnp.float32),
            jax.ShapeDtypeStruct((n, cls), jnp.float32),
        ),
    )(accp, xw, dinv, b, w2)


def _combine_logsoftmax(accp, h2, dinv, b):
    n, cls = h2.shape

    def body(accp_ref, h2_ref, dinv_ref, b_ref, o_ref):
        dv = dinv_ref[...]
        agg = accp_ref[0, :n, :] + accp_ref[1, :n, :]
        z = dv * agg + (dv * dv) * h2_ref[...] + b_ref[...]
        m = jnp.max(z, axis=1, keepdims=True)
        shifted = z - m
        lse = jnp.log(jnp.sum(jnp.exp(shifted), axis=1, keepdims=True))
        o_ref[...] = shifted - lse

    return pl.pallas_call(
        body, out_shape=jax.ShapeDtypeStruct((n, cls), jnp.float32)
    )(accp, h2, dinv, b)


def kernel(x, edge_index, W1, b1, W2, b2):
    n, f_in = x.shape
    e = edge_index.shape[1]
    hid = W1.shape[1]
    cls = W2.shape[1]

    per_block = NW * CHUNK
    ch = -(-e // per_block)
    ch += ch % 2
    epad = ch * per_block
    rps = -(-(n + 1) // (NS * 8)) * 8
    acc_rows = rps * NS

    src = edge_index[0]
    dst = edge_index[1]
    pad = epad - e
    srcp = jnp.concatenate([src, jnp.zeros((pad,), jnp.int32)]).reshape(
        NW, ch, CHUNK
    )
    dstp = jnp.concatenate(
        [dst, jnp.full((pad,), n, jnp.int32)]
    ).reshape(NW, ch, CHUNK)

    zeros16 = jnp.zeros((rps, 16), jnp.float32)
    zeros_cls = jnp.zeros((rps, cls), jnp.float32)

    degp = _make_deg_kernel(ch, acc_rows, rps)(zeros16, dstp)
    xw1 = _matmul(x, W1)
    dinv, xws1 = _dinv_and_scale(degp, xw1)
    acc1 = _make_segsum_kernel(hid, ch, acc_rows, rps)(
        xws1, zeros16, srcp, dstp
    )
    h2, xws2 = _combine_relu_mm(acc1, xw1, dinv, b1.reshape(1, hid), W2)
    acc2 = _make_segsum_kernel(cls, ch, acc_rows, rps)(
        xws2, zeros_cls, srcp, dstp
    )
    return _combine_logsoftmax(acc2, h2, dinv, b2.reshape(1, cls))


# speedup vs baseline: 38.4167x; 1.7365x over previous
import functools

import jax
import jax.numpy as jnp
from jax import lax
from jax.experimental import pallas as pl
from jax.experimental.pallas import tpu as pltpu
from jax.experimental.pallas import tpu_sc as plsc

NC = 2
NS = 16
NW = NC * NS
CHUNK = 128


def _sc_mesh():
    return plsc.VectorSubcoreMesh(
        core_axis_name="c", subcore_axis_name="s", num_cores=NC, num_subcores=NS
    )


@functools.lru_cache(maxsize=None)
def _make_deg_kernel(ch, acc_rows, rps):
    @functools.partial(
        pl.kernel,
        out_type=jax.ShapeDtypeStruct((NC, acc_rows, 16), jnp.float32),
        mesh=_sc_mesh(),
        compiler_params=pltpu.CompilerParams(use_tc_tiling_on_sc=False),
        scratch_types=[
            pltpu.VMEM((ch, CHUNK), jnp.int32),
            pltpu.VMEM((CHUNK, 16), jnp.float32),
            pltpu.VMEM_SHARED((acc_rows, 16), jnp.float32),
            pltpu.SemaphoreType.DMA,
        ],
    )
    def deg_kernel(zeros_hbm, dst_hbm, out_hbm, dst_v, ones_v, acc_sh, sem):
        c = lax.axis_index("c")
        s = lax.axis_index("s")
        w = c * NS + s
        rbase = s * rps
        zcp = pltpu.async_copy(zeros_hbm, acc_sh.at[pl.ds(rbase, rps)], sem)
        @pl.loop(0, CHUNK)
        def _(i):
            ones_v[i, :] = jnp.full((16,), 1.0, jnp.float32)

        pltpu.sync_copy(dst_hbm.at[w], dst_v)
        zcp.wait()
        plsc.subcore_barrier()

        @pl.loop(0, ch)
        def _(j):
            pltpu.sync_copy(ones_v, acc_sh.at[dst_v.at[j]], add=True)

        plsc.subcore_barrier()
        pltpu.sync_copy(
            acc_sh.at[pl.ds(rbase, rps)], out_hbm.at[c, pl.ds(rbase, rps)]
        )

    return deg_kernel


@functools.lru_cache(maxsize=None)
def _make_segsum_kernel(d, ch, acc_rows, rps):
    @functools.partial(
        pl.kernel,
        out_type=jax.ShapeDtypeStruct((NC, acc_rows, d), jnp.float32),
        mesh=_sc_mesh(),
        compiler_params=pltpu.CompilerParams(use_tc_tiling_on_sc=False),
        scratch_types=[
            pltpu.VMEM((ch, CHUNK), jnp.int32),
            pltpu.VMEM((ch, CHUNK), jnp.int32),
            pltpu.VMEM((CHUNK, d), jnp.float32),
            pltpu.VMEM((CHUNK, d), jnp.float32),
            pltpu.VMEM_SHARED((acc_rows, d), jnp.float32),
            pltpu.SemaphoreType.DMA,
            pltpu.SemaphoreType.DMA,
        ],
    )
    def segsum_kernel(
        table_hbm, zeros_hbm, src_hbm, dst_hbm, out_hbm,
        src_v, dst_v, rows_a, rows_b, acc_sh, sem_a, sem_b,
    ):
        c = lax.axis_index("c")
        s = lax.axis_index("s")
        w = c * NS + s
        rbase = s * rps
        zcp = pltpu.async_copy(zeros_hbm, acc_sh.at[pl.ds(rbase, rps)], sem_a)
        pltpu.sync_copy(src_hbm.at[w], src_v)
        pltpu.sync_copy(dst_hbm.at[w], dst_v)
        zcp.wait()
        plsc.subcore_barrier()

        pltpu.async_copy(table_hbm.at[src_v.at[0]], rows_a, sem_a)
        pltpu.async_copy(table_hbm.at[src_v.at[1]], rows_b, sem_b)

        @pl.loop(0, ch, step=2)
        def _(j):
            pltpu.make_async_copy(
                table_hbm.at[src_v.at[j]], rows_a, sem_a
            ).wait()
            pltpu.sync_copy(rows_a, acc_sh.at[dst_v.at[j]], add=True)

            @pl.when(j + 2 < ch)
            def _():
                pltpu.async_copy(table_hbm.at[src_v.at[j + 2]], rows_a, sem_a)

            pltpu.make_async_copy(
                table_hbm.at[src_v.at[j + 1]], rows_b, sem_b
            ).wait()
            pltpu.sync_copy(rows_b, acc_sh.at[dst_v.at[j + 1]], add=True)

            @pl.when(j + 3 < ch)
            def _():
                pltpu.async_copy(table_hbm.at[src_v.at[j + 3]], rows_b, sem_b)

        plsc.subcore_barrier()
        pltpu.sync_copy(
            acc_sh.at[pl.ds(rbase, rps)], out_hbm.at[c, pl.ds(rbase, rps)]
        )

    return segsum_kernel


def _matmul(x, w):
    m, k = x.shape
    k2, n = w.shape

    def body(x_ref, w_ref, o_ref):
        o_ref[...] = jnp.dot(
            x_ref[...], w_ref[...], preferred_element_type=jnp.float32
        )

    return pl.pallas_call(
        body, out_shape=jax.ShapeDtypeStruct((m, n), jnp.float32)
    )(x, w)


def _dinv_and_scale(degp, xw):
    n, hid = xw.shape

    def body(degp_ref, xw_ref, dinv_ref, xws_ref):
        deg = degp_ref[0, :n, 0:1] + degp_ref[1, :n, 0:1] + 1.0
        dinv = lax.rsqrt(deg)
        dinv_ref[...] = dinv
        xws_ref[...] = xw_ref[...] * dinv

    return pl.pallas_call(
        body,
        out_shape=(
            jax.ShapeDtypeStruct((n, 1), jnp.float32),
            jax.ShapeDtypeStruct((n, hid), jnp.float32),
        ),
    )(degp, xw)


def _combine_relu_scale(accp, xw, dinv, b):
    n, hid = xw.shape

    def body(accp_ref, xw_ref, dinv_ref, b_ref, h_ref, hs_ref):
        dv = dinv_ref[...]
        agg = accp_ref[0, :n, :] + accp_ref[1, :n, :]
        h = dv * agg + (dv * dv) * xw_ref[...] + b_ref[...]
        h = jnp.maximum(h, 0.0)
        h_ref[...] = h
        hs_ref[...] = h * dv

    return pl.pallas_call(
        body,
        out_shape=(
            jax.ShapeDtypeStruct((n, hid), jnp.float32),
            jax.ShapeDtypeStruct((n, hid), jnp.float32),
        ),
    )(accp, xw, dinv, b)


def _combine_mm_logsoftmax(accp, h, dinv, w2, b):
    n, hid = h.shape
    cls = w2.shape[1]

    def body(accp_ref, h_ref, dinv_ref, w2_ref, b_ref, o_ref):
        dv = dinv_ref[...]
        agg = accp_ref[0, :n, :] + accp_ref[1, :n, :]
        t = dv * agg + (dv * dv) * h_ref[...]
        z = (
            jnp.dot(t, w2_ref[...], preferred_element_type=jnp.float32)
            + b_ref[...]
        )
        m = jnp.max(z, axis=1, keepdims=True)
        shifted = z - m
        lse = jnp.log(jnp.sum(jnp.exp(shifted), axis=1, keepdims=True))
        o_ref[...] = shifted - lse

    return pl.pallas_call(
        body, out_shape=jax.ShapeDtypeStruct((n, cls), jnp.float32)
    )(accp, h, dinv, w2, b)


def kernel(x, edge_index, W1, b1, W2, b2):
    n, f_in = x.shape
    e = edge_index.shape[1]
    hid = W1.shape[1]
    cls = W2.shape[1]

    per_block = NW * CHUNK
    ch = -(-e // per_block)
    ch += ch % 2
    epad = ch * per_block
    rps = -(-(n + 1) // (NS * 8)) * 8
    acc_rows = rps * NS

    src = edge_index[0]
    dst = edge_index[1]
    pad = epad - e
    srcp = jnp.concatenate([src, jnp.zeros((pad,), jnp.int32)]).reshape(
        NW, ch, CHUNK
    )
    dstp = jnp.concatenate(
        [dst, jnp.full((pad,), n, jnp.int32)]
    ).reshape(NW, ch, CHUNK)

    zeros16 = jnp.zeros((rps, 16), jnp.float32)

    degp = _make_deg_kernel(ch, acc_rows, rps)(zeros16, dstp)
    xw1 = _matmul(x, W1)
    dinv, xws1 = _dinv_and_scale(degp, xw1)
    acc1 = _make_segsum_kernel(hid, ch, acc_rows, rps)(
        xws1, zeros16, srcp, dstp
    )
    h, hs = _combine_relu_scale(acc1, xw1, dinv, b1.reshape(1, hid))
    acc2 = _make_segsum_kernel(hid, ch, acc_rows, rps)(
        hs, zeros16, srcp, dstp
    )
    return _combine_mm_logsoftmax(acc2, h, dinv, W2, b2.reshape(1, cls))


# speedup vs baseline: 55.8170x; 1.4529x over previous
import functools

import jax
import jax.numpy as jnp
from jax import lax
from jax.experimental import pallas as pl
from jax.experimental.pallas import tpu as pltpu
from jax.experimental.pallas import tpu_sc as plsc

NC = 2
NS = 16
NW = NC * NS
CHUNK = 128


def _sc_mesh():
    return plsc.VectorSubcoreMesh(
        core_axis_name="c", subcore_axis_name="s", num_cores=NC, num_subcores=NS
    )


@functools.lru_cache(maxsize=None)
def _make_deg_kernel(ch, acc_rows, rps):
    @functools.partial(
        pl.kernel,
        out_type=jax.ShapeDtypeStruct((NC, acc_rows, 16), jnp.float32),
        mesh=_sc_mesh(),
        compiler_params=pltpu.CompilerParams(use_tc_tiling_on_sc=False),
        scratch_types=[
            pltpu.VMEM((ch, CHUNK), jnp.int32),
            pltpu.VMEM((CHUNK, 16), jnp.float32),
            pltpu.VMEM_SHARED((acc_rows, 16), jnp.float32),
            pltpu.SemaphoreType.DMA,
        ],
    )
    def deg_kernel(zeros_hbm, ones_hbm, dst_hbm, out_hbm, dst_v, ones_v,
                   acc_sh, sem):
        c = lax.axis_index("c")
        s = lax.axis_index("s")
        w = c * NS + s
        rbase = s * rps
        zcp = pltpu.async_copy(zeros_hbm, acc_sh.at[pl.ds(rbase, rps)], sem)
        pltpu.sync_copy(ones_hbm, ones_v)
        pltpu.sync_copy(dst_hbm.at[w], dst_v)
        zcp.wait()
        plsc.subcore_barrier()

        @pl.loop(0, ch)
        def _(j):
            pltpu.sync_copy(ones_v, acc_sh.at[dst_v.at[j]], add=True)

        plsc.subcore_barrier()
        pltpu.sync_copy(
            acc_sh.at[pl.ds(rbase, rps)], out_hbm.at[c, pl.ds(rbase, rps)]
        )

    return deg_kernel


@functools.lru_cache(maxsize=None)
def _make_segsum_kernel(d, ch, acc_rows, rps, n_tab):
    assert n_tab % NS == 0
    tps = n_tab // NS

    @functools.partial(
        pl.kernel,
        out_type=jax.ShapeDtypeStruct((NC, acc_rows, d), jnp.float32),
        mesh=_sc_mesh(),
        compiler_params=pltpu.CompilerParams(use_tc_tiling_on_sc=False),
        scratch_types=[
            pltpu.VMEM((ch, CHUNK), jnp.int32),
            pltpu.VMEM((ch, CHUNK), jnp.int32),
            pltpu.VMEM((CHUNK, d), jnp.float32),
            pltpu.VMEM((CHUNK, d), jnp.float32),
            pltpu.VMEM_SHARED((n_tab, d), jnp.float32),
            pltpu.VMEM_SHARED((acc_rows, d), jnp.float32),
            pltpu.SemaphoreType.DMA,
            pltpu.SemaphoreType.DMA,
        ],
    )
    def segsum_kernel(
        table_hbm, zeros_hbm, src_hbm, dst_hbm, out_hbm,
        src_v, dst_v, rows_a, rows_b, tab_sh, acc_sh, sem_a, sem_b,
    ):
        c = lax.axis_index("c")
        s = lax.axis_index("s")
        w = c * NS + s
        rbase = s * rps
        tbase = s * tps
        tcp = pltpu.async_copy(
            table_hbm.at[pl.ds(tbase, tps)], tab_sh.at[pl.ds(tbase, tps)], sem_b
        )
        zcp = pltpu.async_copy(zeros_hbm, acc_sh.at[pl.ds(rbase, rps)], sem_a)
        pltpu.sync_copy(src_hbm.at[w], src_v)
        pltpu.sync_copy(dst_hbm.at[w], dst_v)
        zcp.wait()
        tcp.wait()
        plsc.subcore_barrier()

        pltpu.async_copy(tab_sh.at[src_v.at[0]], rows_a, sem_a)
        pltpu.async_copy(tab_sh.at[src_v.at[1]], rows_b, sem_b)

        @pl.loop(0, ch, step=2)
        def _(j):
            pltpu.make_async_copy(
                tab_sh.at[src_v.at[j]], rows_a, sem_a
            ).wait()
            pltpu.sync_copy(rows_a, acc_sh.at[dst_v.at[j]], add=True)

            @pl.when(j + 2 < ch)
            def _():
                pltpu.async_copy(tab_sh.at[src_v.at[j + 2]], rows_a, sem_a)

            pltpu.make_async_copy(
                tab_sh.at[src_v.at[j + 1]], rows_b, sem_b
            ).wait()
            pltpu.sync_copy(rows_b, acc_sh.at[dst_v.at[j + 1]], add=True)

            @pl.when(j + 3 < ch)
            def _():
                pltpu.async_copy(tab_sh.at[src_v.at[j + 3]], rows_b, sem_b)

        plsc.subcore_barrier()
        pltpu.sync_copy(
            acc_sh.at[pl.ds(rbase, rps)], out_hbm.at[c, pl.ds(rbase, rps)]
        )

    return segsum_kernel


def _matmul(x, w):
    m, k = x.shape
    k2, n = w.shape

    def body(x_ref, w_ref, o_ref):
        o_ref[...] = jnp.dot(
            x_ref[...], w_ref[...], preferred_element_type=jnp.float32
        )

    return pl.pallas_call(
        body, out_shape=jax.ShapeDtypeStruct((m, n), jnp.float32)
    )(x, w)


def _dinv_and_scale(degp, xw):
    n, hid = xw.shape

    def body(degp_ref, xw_ref, dinv_ref, xws_ref):
        deg = degp_ref[0, :n, 0:1] + degp_ref[1, :n, 0:1] + 1.0
        dinv = lax.rsqrt(deg)
        dinv_ref[...] = dinv
        xws_ref[...] = xw_ref[...] * dinv

    return pl.pallas_call(
        body,
        out_shape=(
            jax.ShapeDtypeStruct((n, 1), jnp.float32),
            jax.ShapeDtypeStruct((n, hid), jnp.float32),
        ),
    )(degp, xw)


def _combine_relu_scale(accp, xw, dinv, b):
    n, hid = xw.shape

    def body(accp_ref, xw_ref, dinv_ref, b_ref, h_ref, hs_ref):
        dv = dinv_ref[...]
        agg = accp_ref[0, :n, :] + accp_ref[1, :n, :]
        h = dv * agg + (dv * dv) * xw_ref[...] + b_ref[...]
        h = jnp.maximum(h, 0.0)
        h_ref[...] = h
        hs_ref[...] = h * dv

    return pl.pallas_call(
        body,
        out_shape=(
            jax.ShapeDtypeStruct((n, hid), jnp.float32),
            jax.ShapeDtypeStruct((n, hid), jnp.float32),
        ),
    )(accp, xw, dinv, b)


def _combine_mm_logsoftmax(accp, h, dinv, w2, b):
    n, hid = h.shape
    cls = w2.shape[1]

    def body(accp_ref, h_ref, dinv_ref, w2_ref, b_ref, o_ref):
        dv = dinv_ref[...]
        agg = accp_ref[0, :n, :] + accp_ref[1, :n, :]
        t = dv * agg + (dv * dv) * h_ref[...]
        z = (
            jnp.dot(t, w2_ref[...], preferred_element_type=jnp.float32)
            + b_ref[...]
        )
        m = jnp.max(z, axis=1, keepdims=True)
        shifted = z - m
        lse = jnp.log(jnp.sum(jnp.exp(shifted), axis=1, keepdims=True))
        o_ref[...] = shifted - lse

    return pl.pallas_call(
        body, out_shape=jax.ShapeDtypeStruct((n, cls), jnp.float32)
    )(accp, h, dinv, w2, b)


def kernel(x, edge_index, W1, b1, W2, b2):
    n, f_in = x.shape
    e = edge_index.shape[1]
    hid = W1.shape[1]
    cls = W2.shape[1]

    per_block = NW * CHUNK
    ch = -(-e // per_block)
    ch += ch % 2
    epad = ch * per_block
    rps = -(-(n + 1) // (NS * 8)) * 8
    acc_rows = rps * NS

    src = edge_index[0]
    dst = edge_index[1]
    pad = epad - e
    srcp = jnp.concatenate([src, jnp.zeros((pad,), jnp.int32)]).reshape(
        NW, ch, CHUNK
    )
    spare = acc_rows - n
    dstp = jnp.concatenate(
        [dst, n + (jnp.arange(pad, dtype=jnp.int32) % spare)]
    ).reshape(NW, ch, CHUNK)

    zeros16 = jnp.zeros((rps, 16), jnp.float32)
    ones16 = jnp.ones((CHUNK, 16), jnp.float32)

    degp = _make_deg_kernel(ch, acc_rows, rps)(zeros16, ones16, dstp)
    xw1 = _matmul(x, W1)
    dinv, xws1 = _dinv_and_scale(degp, xw1)
    acc1 = _make_segsum_kernel(hid, ch, acc_rows, rps, n)(
        xws1, zeros16, srcp, dstp
    )
    h, hs = _combine_relu_scale(acc1, xw1, dinv, b1.reshape(1, hid))
    acc2 = _make_segsum_kernel(hid, ch, acc_rows, rps, n)(
        hs, zeros16, srcp, dstp
    )
    return _combine_mm_logsoftmax(acc2, h, dinv, W2, b2.reshape(1, cls))


# speedup vs baseline: 60.2513x; 1.0794x over previous
import functools

import jax
import jax.numpy as jnp
from jax import lax
from jax.experimental import pallas as pl
from jax.experimental.pallas import tpu as pltpu
from jax.experimental.pallas import tpu_sc as plsc

NC = 2
NS = 16
NW = NC * NS
CHUNK = 128
BN = 1000


def _sc_mesh():
    return plsc.VectorSubcoreMesh(
        core_axis_name="c", subcore_axis_name="s", num_cores=NC, num_subcores=NS
    )


def _worker_range(w, nch_tot):
    nfull, rem = nch_tot // NW, nch_tot % NW
    base = jnp.where(w < rem, w * (nfull + 1), w * nfull + rem)
    nch = jnp.where(w < rem, nfull + 1, nfull)
    return base, nch, nfull, rem


@functools.lru_cache(maxsize=None)
def _make_deg_kernel(nch_tot, acc_rows, rps):
    nfull = nch_tot // NW
    slab = nfull + (1 if nch_tot % NW else 0)

    @functools.partial(
        pl.kernel,
        out_type=jax.ShapeDtypeStruct((NC, acc_rows, 16), jnp.float32),
        mesh=_sc_mesh(),
        compiler_params=pltpu.CompilerParams(use_tc_tiling_on_sc=False),
        scratch_types=[
            pltpu.VMEM((slab, CHUNK), jnp.int32),
            pltpu.VMEM((CHUNK, 16), jnp.float32),
            pltpu.VMEM_SHARED((acc_rows, 16), jnp.float32),
            pltpu.SemaphoreType.DMA,
        ],
    )
    def deg_kernel(zeros_hbm, ones_hbm, eidx_hbm, out_hbm, dst_v, ones_v,
                   acc_sh, sem):
        c = lax.axis_index("c")
        s = lax.axis_index("s")
        w = c * NS + s
        rbase = s * rps
        base, nch, _, rem = _worker_range(w, nch_tot)
        zcp = pltpu.async_copy(zeros_hbm, acc_sh.at[pl.ds(rbase, rps)], sem)
        pltpu.sync_copy(ones_hbm, ones_v)
        if nch_tot % NW:
            @pl.when(w < rem)
            def _():
                pltpu.sync_copy(eidx_hbm.at[1, pl.ds(base, slab)], dst_v)

            @pl.when(w >= rem)
            def _():
                pltpu.sync_copy(
                    eidx_hbm.at[1, pl.ds(base, nfull)],
                    dst_v.at[pl.ds(0, nfull)],
                )
        else:
            pltpu.sync_copy(eidx_hbm.at[1, pl.ds(base, slab)], dst_v)
        zcp.wait()
        plsc.subcore_barrier()

        @pl.loop(0, nch)
        def _(j):
            pltpu.sync_copy(ones_v, acc_sh.at[dst_v.at[j]], add=True)

        plsc.subcore_barrier()
        pltpu.sync_copy(
            acc_sh.at[pl.ds(rbase, rps)], out_hbm.at[c, pl.ds(rbase, rps)]
        )

    return deg_kernel


@functools.lru_cache(maxsize=None)
def _make_segsum_kernel(d, nch_tot, acc_rows, rps, n_tab):
    assert n_tab % NS == 0
    tps = n_tab // NS
    nfull = nch_tot // NW
    assert nfull >= 2 and nfull % 2 == 0
    slab = nfull + (1 if nch_tot % NW else 0)

    @functools.partial(
        pl.kernel,
        out_type=jax.ShapeDtypeStruct((NC, acc_rows, d), jnp.float32),
        mesh=_sc_mesh(),
        compiler_params=pltpu.CompilerParams(use_tc_tiling_on_sc=False),
        scratch_types=[
            pltpu.VMEM((slab, CHUNK), jnp.int32),
            pltpu.VMEM((slab, CHUNK), jnp.int32),
            pltpu.VMEM((CHUNK, d), jnp.float32),
            pltpu.VMEM((CHUNK, d), jnp.float32),
            pltpu.VMEM_SHARED((n_tab, d), jnp.float32),
            pltpu.VMEM_SHARED((acc_rows, d), jnp.float32),
            pltpu.SemaphoreType.DMA,
            pltpu.SemaphoreType.DMA,
        ],
    )
    def segsum_kernel(
        table_hbm, zeros_hbm, eidx_hbm, out_hbm,
        src_v, dst_v, rows_a, rows_b, tab_sh, acc_sh, sem_a, sem_b,
    ):
        c = lax.axis_index("c")
        s = lax.axis_index("s")
        w = c * NS + s
        rbase = s * rps
        base, _, _, rem = _worker_range(w, nch_tot)
        tbase = s * tps
        tcp = pltpu.async_copy(
            table_hbm.at[pl.ds(tbase, tps)], tab_sh.at[pl.ds(tbase, tps)], sem_b
        )
        zcp = pltpu.async_copy(zeros_hbm, acc_sh.at[pl.ds(rbase, rps)], sem_a)
        if nch_tot % NW:
            @pl.when(w < rem)
            def _():
                pltpu.sync_copy(eidx_hbm.at[0, pl.ds(base, slab)], src_v)
                pltpu.sync_copy(eidx_hbm.at[1, pl.ds(base, slab)], dst_v)

            @pl.when(w >= rem)
            def _():
                pltpu.sync_copy(
                    eidx_hbm.at[0, pl.ds(base, nfull)],
                    src_v.at[pl.ds(0, nfull)],
                )
                pltpu.sync_copy(
                    eidx_hbm.at[1, pl.ds(base, nfull)],
                    dst_v.at[pl.ds(0, nfull)],
                )
        else:
            pltpu.sync_copy(eidx_hbm.at[0, pl.ds(base, slab)], src_v)
            pltpu.sync_copy(eidx_hbm.at[1, pl.ds(base, slab)], dst_v)
        zcp.wait()
        tcp.wait()
        plsc.subcore_barrier()

        pltpu.async_copy(tab_sh.at[src_v.at[0]], rows_a, sem_a)
        pltpu.async_copy(tab_sh.at[src_v.at[1]], rows_b, sem_b)

        @pl.loop(0, nfull, step=2)
        def _(j):
            pltpu.make_async_copy(
                tab_sh.at[src_v.at[j]], rows_a, sem_a
            ).wait()
            pltpu.sync_copy(rows_a, acc_sh.at[dst_v.at[j]], add=True)

            @pl.when(j + 2 < nfull)
            def _():
                pltpu.async_copy(tab_sh.at[src_v.at[j + 2]], rows_a, sem_a)

            pltpu.make_async_copy(
                tab_sh.at[src_v.at[j + 1]], rows_b, sem_b
            ).wait()
            pltpu.sync_copy(rows_b, acc_sh.at[dst_v.at[j + 1]], add=True)

            @pl.when(j + 3 < nfull)
            def _():
                pltpu.async_copy(tab_sh.at[src_v.at[j + 3]], rows_b, sem_b)

        if nch_tot % NW:
            @pl.when(w < rem)
            def _():
                pltpu.async_copy(
                    tab_sh.at[src_v.at[nfull]], rows_a, sem_a
                ).wait()
                pltpu.sync_copy(rows_a, acc_sh.at[dst_v.at[nfull]], add=True)

        plsc.subcore_barrier()
        pltpu.sync_copy(
            acc_sh.at[pl.ds(rbase, rps)], out_hbm.at[c, pl.ds(rbase, rps)]
        )

    return segsum_kernel


def _matmul(x, w):
    m, k = x.shape
    _, nn = w.shape
    assert m % BN == 0

    def body(x_ref, w_ref, o_ref):
        o_ref[...] = jnp.dot(
            x_ref[...], w_ref[...], preferred_element_type=jnp.float32
        )

    return pl.pallas_call(
        body,
        grid=(m // BN,),
        in_specs=[
            pl.BlockSpec((BN, k), lambda i: (i, 0)),
            pl.BlockSpec((k, nn), lambda i: (0, 0)),
        ],
        out_specs=pl.BlockSpec((BN, nn), lambda i: (i, 0)),
        out_shape=jax.ShapeDtypeStruct((m, nn), jnp.float32),
    )(x, w)


def _dinv_and_scale(degp, xw):
    n, hid = xw.shape
    rows = degp.shape[1]

    def body(degp_ref, xw_ref, dinv_ref, xws_ref):
        deg = degp_ref[0, :, 0:1] + degp_ref[1, :, 0:1] + 1.0
        dinv = lax.rsqrt(deg)
        dinv_ref[...] = dinv
        xws_ref[...] = xw_ref[...] * dinv

    return pl.pallas_call(
        body,
        grid=(n // BN,),
        in_specs=[
            pl.BlockSpec((2, BN, 16), lambda i: (0, i, 0)),
            pl.BlockSpec((BN, hid), lambda i: (i, 0)),
        ],
        out_specs=(
            pl.BlockSpec((BN, 1), lambda i: (i, 0)),
            pl.BlockSpec((BN, hid), lambda i: (i, 0)),
        ),
        out_shape=(
            jax.ShapeDtypeStruct((n, 1), jnp.float32),
            jax.ShapeDtypeStruct((n, hid), jnp.float32),
        ),
    )(degp, xw)


def _combine_relu_scale(accp, xw, dinv, b):
    n, hid = xw.shape

    def body(accp_ref, xw_ref, dinv_ref, b_ref, h_ref, hs_ref):
        dv = dinv_ref[...]
        agg = accp_ref[0, :, :] + accp_ref[1, :, :]
        h = dv * agg + (dv * dv) * xw_ref[...] + b_ref[...]
        h = jnp.maximum(h, 0.0)
        h_ref[...] = h
        hs_ref[...] = h * dv

    return pl.pallas_call(
        body,
        grid=(n // BN,),
        in_specs=[
            pl.BlockSpec((2, BN, hid), lambda i: (0, i, 0)),
            pl.BlockSpec((BN, hid), lambda i: (i, 0)),
            pl.BlockSpec((BN, 1), lambda i: (i, 0)),
            pl.BlockSpec((1, hid), lambda i: (0, 0)),
        ],
        out_specs=(
            pl.BlockSpec((BN, hid), lambda i: (i, 0)),
            pl.BlockSpec((BN, hid), lambda i: (i, 0)),
        ),
        out_shape=(
            jax.ShapeDtypeStruct((n, hid), jnp.float32),
            jax.ShapeDtypeStruct((n, hid), jnp.float32),
        ),
    )(accp, xw, dinv, b)


def _combine_mm_logsoftmax(accp, h, dinv, w2, b):
    n, hid = h.shape
    cls = w2.shape[1]

    def body(accp_ref, h_ref, dinv_ref, w2_ref, b_ref, o_ref):
        dv = dinv_ref[...]
        agg = accp_ref[0, :, :] + accp_ref[1, :, :]
        t = dv * agg + (dv * dv) * h_ref[...]
        z = (
            jnp.dot(t, w2_ref[...], preferred_element_type=jnp.float32)
            + b_ref[...]
        )
        m = jnp.max(z, axis=1, keepdims=True)
        shifted = z - m
        lse = jnp.log(jnp.sum(jnp.exp(shifted), axis=1, keepdims=True))
        o_ref[...] = shifted - lse

    return pl.pallas_call(
        body,
        grid=(n // BN,),
        in_specs=[
            pl.BlockSpec((2, BN, hid), lambda i: (0, i, 0)),
            pl.BlockSpec((BN, hid), lambda i: (i, 0)),
            pl.BlockSpec((BN, 1), lambda i: (i, 0)),
            pl.BlockSpec((hid, cls), lambda i: (0, 0)),
            pl.BlockSpec((1, cls), lambda i: (0, 0)),
        ],
        out_specs=pl.BlockSpec((BN, cls), lambda i: (i, 0)),
        out_shape=jax.ShapeDtypeStruct((n, cls), jnp.float32),
    )(accp, h, dinv, w2, b)


def kernel(x, edge_index, W1, b1, W2, b2):
    n, f_in = x.shape
    e = edge_index.shape[1]
    hid = W1.shape[1]
    cls = W2.shape[1]

    if e % CHUNK:
        pad = CHUNK - e % CHUNK
        pad_edges = jnp.concatenate(
            [jnp.zeros((1, pad), jnp.int32),
             jnp.full((1, pad), n, jnp.int32)],
            axis=0,
        )
        edge_index = jnp.concatenate([edge_index, pad_edges], axis=1)
        e += pad
    nch_tot = e // CHUNK
    eidx = edge_index.reshape(2, nch_tot, CHUNK)

    rps = -(-(n + 1) // (NS * 8)) * 8
    acc_rows = rps * NS

    zeros16 = jnp.zeros((rps, 16), jnp.float32)
    ones16 = jnp.ones((CHUNK, 16), jnp.float32)

    degp = _make_deg_kernel(nch_tot, acc_rows, rps)(zeros16, ones16, eidx)
    xw1 = _matmul(x, W1)
    dinv, xws1 = _dinv_and_scale(degp, xw1)
    acc1 = _make_segsum_kernel(hid, nch_tot, acc_rows, rps, n)(
        xws1, zeros16, eidx
    )
    h, hs = _combine_relu_scale(acc1, xw1, dinv, b1.reshape(1, hid))
    acc2 = _make_segsum_kernel(hid, nch_tot, acc_rows, rps, n)(
        hs, zeros16, eidx
    )
    return _combine_mm_logsoftmax(acc2, h, dinv, W2, b2.reshape(1, cls))


# speedup vs baseline: 84.9355x; 1.4097x over previous
import functools

import jax
import jax.numpy as jnp
from jax import lax
from jax.experimental import pallas as pl
from jax.experimental.pallas import tpu as pltpu
from jax.experimental.pallas import tpu_sc as plsc

NC = 2
NS = 16
NW = NC * NS
CHUNK = 128
BN = 1000


def _sc_mesh():
    return plsc.VectorSubcoreMesh(
        core_axis_name="c", subcore_axis_name="s", num_cores=NC, num_subcores=NS
    )


def _worker_range(w, nch_tot):
    nfull, rem = nch_tot // NW, nch_tot % NW
    base = jnp.where(w < rem, w * (nfull + 1), w * nfull + rem)
    nch = jnp.where(w < rem, nfull + 1, nfull)
    return base, nch, nfull, rem


@functools.lru_cache(maxsize=None)
def _make_deg_kernel(nch_tot, acc_rows, rps):
    nfull = nch_tot // NW
    slab = nfull + (1 if nch_tot % NW else 0)

    @functools.partial(
        pl.kernel,
        out_type=jax.ShapeDtypeStruct((NC, acc_rows, 16), jnp.float32),
        mesh=_sc_mesh(),
        compiler_params=pltpu.CompilerParams(use_tc_tiling_on_sc=False),
        scratch_types=[
            pltpu.VMEM((slab, CHUNK), jnp.int32),
            pltpu.VMEM((CHUNK, 16), jnp.float32),
            pltpu.VMEM_SHARED((acc_rows, 16), jnp.float32),
            pltpu.SemaphoreType.DMA,
        ],
    )
    def deg_kernel(zeros_hbm, ones_hbm, eidx_hbm, out_hbm, dst_v, ones_v,
                   acc_sh, sem):
        c = lax.axis_index("c")
        s = lax.axis_index("s")
        w = c * NS + s
        rbase = s * rps
        base, nch, _, rem = _worker_range(w, nch_tot)
        zcp = pltpu.async_copy(zeros_hbm, acc_sh.at[pl.ds(rbase, rps)], sem)
        pltpu.sync_copy(ones_hbm, ones_v)
        if nch_tot % NW:
            @pl.when(w < rem)
            def _():
                pltpu.sync_copy(eidx_hbm.at[1, pl.ds(base, slab)], dst_v)

            @pl.when(w >= rem)
            def _():
                pltpu.sync_copy(
                    eidx_hbm.at[1, pl.ds(base, nfull)],
                    dst_v.at[pl.ds(0, nfull)],
                )
        else:
            pltpu.sync_copy(eidx_hbm.at[1, pl.ds(base, slab)], dst_v)
        zcp.wait()
        plsc.subcore_barrier()

        @pl.loop(0, nch)
        def _(j):
            pltpu.sync_copy(ones_v, acc_sh.at[dst_v.at[j]], add=True)

        plsc.subcore_barrier()
        pltpu.sync_copy(
            acc_sh.at[pl.ds(rbase, rps)], out_hbm.at[c, pl.ds(rbase, rps)]
        )

    return deg_kernel


@functools.lru_cache(maxsize=None)
def _make_segsum_kernel(d, nch_tot, acc_rows, rps, n_tab):
    assert n_tab % NS == 0
    tps = n_tab // NS
    nfull = nch_tot // NW
    assert nfull >= 2 and nfull % 2 == 0
    slab = nfull + (1 if nch_tot % NW else 0)

    @functools.partial(
        pl.kernel,
        out_type=jax.ShapeDtypeStruct((NC, acc_rows, d), jnp.float32),
        mesh=_sc_mesh(),
        compiler_params=pltpu.CompilerParams(use_tc_tiling_on_sc=False),
        scratch_types=[
            pltpu.VMEM((slab, CHUNK), jnp.int32),
            pltpu.VMEM((slab, CHUNK), jnp.int32),
            pltpu.VMEM((CHUNK, d), jnp.float32),
            pltpu.VMEM((CHUNK, d), jnp.float32),
            pltpu.VMEM_SHARED((n_tab, d), jnp.float32),
            pltpu.VMEM_SHARED((acc_rows, d), jnp.float32),
            pltpu.SemaphoreType.DMA,
            pltpu.SemaphoreType.DMA,
        ],
    )
    def segsum_kernel(
        table_hbm, zeros_hbm, eidx_hbm, out_hbm,
        src_v, dst_v, rows_a, rows_b, tab_sh, acc_sh, sem_a, sem_b,
    ):
        c = lax.axis_index("c")
        s = lax.axis_index("s")
        w = c * NS + s
        rbase = s * rps
        base, _, _, rem = _worker_range(w, nch_tot)
        tbase = s * tps
        tcp = pltpu.async_copy(
            table_hbm.at[pl.ds(tbase, tps)], tab_sh.at[pl.ds(tbase, tps)], sem_b
        )
        zcp = pltpu.async_copy(zeros_hbm, acc_sh.at[pl.ds(rbase, rps)], sem_a)
        if nch_tot % NW:
            @pl.when(w < rem)
            def _():
                pltpu.sync_copy(eidx_hbm.at[0, pl.ds(base, slab)], src_v)
                pltpu.sync_copy(eidx_hbm.at[1, pl.ds(base, slab)], dst_v)

            @pl.when(w >= rem)
            def _():
                pltpu.sync_copy(
                    eidx_hbm.at[0, pl.ds(base, nfull)],
                    src_v.at[pl.ds(0, nfull)],
                )
                pltpu.sync_copy(
                    eidx_hbm.at[1, pl.ds(base, nfull)],
                    dst_v.at[pl.ds(0, nfull)],
                )
        else:
            pltpu.sync_copy(eidx_hbm.at[0, pl.ds(base, slab)], src_v)
            pltpu.sync_copy(eidx_hbm.at[1, pl.ds(base, slab)], dst_v)
        zcp.wait()
        tcp.wait()
        plsc.subcore_barrier()

        pltpu.async_copy(tab_sh.at[src_v.at[0]], rows_a, sem_a)
        pltpu.async_copy(tab_sh.at[src_v.at[1]], rows_b, sem_b)

        @pl.loop(0, nfull, step=2)
        def _(j):
            pltpu.make_async_copy(
                tab_sh.at[src_v.at[j]], rows_a, sem_a
            ).wait()
            pltpu.sync_copy(rows_a, acc_sh.at[dst_v.at[j]], add=True)

            @pl.when(j + 2 < nfull)
            def _():
                pltpu.async_copy(tab_sh.at[src_v.at[j + 2]], rows_a, sem_a)

            pltpu.make_async_copy(
                tab_sh.at[src_v.at[j + 1]], rows_b, sem_b
            ).wait()
            pltpu.sync_copy(rows_b, acc_sh.at[dst_v.at[j + 1]], add=True)

            @pl.when(j + 3 < nfull)
            def _():
                pltpu.async_copy(tab_sh.at[src_v.at[j + 3]], rows_b, sem_b)

        if nch_tot % NW:
            @pl.when(w < rem)
            def _():
                pltpu.async_copy(
                    tab_sh.at[src_v.at[nfull]], rows_a, sem_a
                ).wait()
                pltpu.sync_copy(rows_a, acc_sh.at[dst_v.at[nfull]], add=True)

        plsc.subcore_barrier()
        pltpu.sync_copy(
            acc_sh.at[pl.ds(rbase, rps)], out_hbm.at[c, pl.ds(rbase, rps)]
        )

    return segsum_kernel


def _matmul_packed(x3, w):
    rows, pk, f_in = x3.shape
    hid = w.shape[1]

    def body(x3_ref, w_ref, o_ref):
        wv = w_ref[...]
        parts = [
            jnp.dot(
                x3_ref[:, k, :], wv, preferred_element_type=jnp.float32
            )
            for k in range(pk)
        ]
        o_ref[...] = jnp.concatenate(parts, axis=1)

    return pl.pallas_call(
        body,
        out_shape=jax.ShapeDtypeStruct((rows, pk * hid), jnp.float32),
    )(x3, w)


def _dinv_and_scale(degp, xwp):
    rows = xwp.shape[0]

    def body(degp_ref, xw_ref, dinv_ref, xws_ref):
        deg = degp_ref[0, :rows, :] + degp_ref[1, :rows, :] + 1.0
        dinv = lax.rsqrt(deg)
        dinv_ref[...] = dinv
        xws_ref[...] = xw_ref[...] * dinv

    return pl.pallas_call(
        body,
        out_shape=(
            jax.ShapeDtypeStruct((rows, 128), jnp.float32),
            jax.ShapeDtypeStruct((rows, 128), jnp.float32),
        ),
    )(degp, xwp)


def _combine_relu_scale(accp, xwp, dinvp, bt):
    rows = xwp.shape[0]

    def body(accp_ref, xw_ref, dinv_ref, b_ref, hs_ref, dh2_ref):
        dv = dinv_ref[...]
        agg = accp_ref[0, :rows, :] + accp_ref[1, :rows, :]
        h = dv * agg + (dv * dv) * xw_ref[...] + b_ref[...]
        h = jnp.maximum(h, 0.0)
        hs_ref[...] = h * dv
        dh2_ref[...] = h * (dv * dv)

    return pl.pallas_call(
        body,
        out_shape=(
            jax.ShapeDtypeStruct((rows, 128), jnp.float32),
            jax.ShapeDtypeStruct((rows, 128), jnp.float32),
        ),
    )(accp, xwp, dinvp, bt)


def _final_mm_logsoftmax(accp, dh2p, dinvp, w2, b):
    rows = dh2p.shape[0]
    hid = w2.shape[0]
    cls = w2.shape[1]
    pk = 128 // hid

    def body(accp_ref, dh2_ref, dinv_ref, w2_ref, b_ref, *outs):
        dv = dinv_ref[...]
        agg = accp_ref[0, :rows, :] + accp_ref[1, :rows, :]
        t = dv * agg + dh2_ref[...]
        wv = w2_ref[...]
        bv = b_ref[...]
        for k in range(pk):
            z = (
                jnp.dot(
                    t[:, k * hid:(k + 1) * hid], wv,
                    preferred_element_type=jnp.float32,
                )
                + bv
            )
            m = jnp.max(z, axis=1, keepdims=True)
            shifted = z - m
            lse = jnp.log(jnp.sum(jnp.exp(shifted), axis=1, keepdims=True))
            outs[k][...] = shifted - lse

    return pl.pallas_call(
        body,
        out_shape=tuple(
            jax.ShapeDtypeStruct((rows, cls), jnp.float32) for _ in range(pk)
        ),
    )(accp, dh2p, dinvp, w2, b)


def kernel(x, edge_index, W1, b1, W2, b2):
    n, f_in = x.shape
    e = edge_index.shape[1]
    hid = W1.shape[1]
    cls = W2.shape[1]

    if e % CHUNK:
        pad = CHUNK - e % CHUNK
        pad_edges = jnp.concatenate(
            [jnp.zeros((1, pad), jnp.int32),
             jnp.full((1, pad), n, jnp.int32)],
            axis=0,
        )
        edge_index = jnp.concatenate([edge_index, pad_edges], axis=1)
        e += pad
    nch_tot = e // CHUNK
    eidx = edge_index.reshape(2, nch_tot, CHUNK)

    rps = -(-(n + 1) // (NS * 8)) * 8
    acc_rows = rps * NS

    zeros16 = jnp.zeros((rps, 16), jnp.float32)
    ones16 = jnp.ones((CHUNK, 16), jnp.float32)

    pk = 128 // hid
    prows = n // pk
    arows = acc_rows // pk
    x3 = x.reshape(prows, pk, f_in)
    b1t = jnp.tile(b1, pk).reshape(1, 128)

    degp = _make_deg_kernel(nch_tot, acc_rows, rps)(zeros16, ones16, eidx)
    xw1p = _matmul_packed(x3, W1)
    dinvp, xws1p = _dinv_and_scale(degp.reshape(NC, arows, 128), xw1p)
    acc1 = _make_segsum_kernel(hid, nch_tot, acc_rows, rps, n)(
        xws1p.reshape(n, hid), zeros16, eidx
    )
    hsp, dh2p = _combine_relu_scale(
        acc1.reshape(NC, arows, 128), xw1p, dinvp, b1t
    )
    acc2 = _make_segsum_kernel(hid, nch_tot, acc_rows, rps, n)(
        hsp.reshape(n, hid), zeros16, eidx
    )
    outs = _final_mm_logsoftmax(
        acc2.reshape(NC, arows, 128), dh2p, dinvp, W2, b2.reshape(1, cls)
    )
    return jnp.stack(outs, axis=1).reshape(n, cls)
